# baseline (speedup 1.0000x reference)
import functools

import jax
import jax.numpy as jnp
from jax import lax
from jax.experimental import pallas as pl
from jax.experimental.pallas import tpu as pltpu
from jax.experimental.pallas import tpu_sc as plsc

_D = 32
_L = 16
_NC = 2
_NS = 16
_NW = _NC * _NS
_TS = 8
_TB = 128


@functools.lru_cache(maxsize=None)
def _gather_call(bsz, seq):
    nst = seq // _TS
    ndt = _D // _TS
    mesh = plsc.VectorSubcoreMesh(core_axis_name="c", subcore_axis_name="s")

    @functools.partial(
        pl.kernel,
        mesh=mesh,
        out_type=jax.ShapeDtypeStruct((seq, ndt, _NW, _TS, _TB), jnp.float32),
        scratch_types=[
            pltpu.VMEM((_TS, _TB), jnp.int32),
            pltpu.VMEM((_TS, _TB, _D), jnp.float32),
            pltpu.VMEM((_TS, ndt, _TS, _TB), jnp.float32),
            pltpu.SemaphoreType.DMA,
        ],
        compiler_params=pltpu.CompilerParams(
            use_tc_tiling_on_sc=False, needs_layout_passes=False),
    )
    def k(idx4_hbm, tab_hbm, out5_hbm, idx_t, rows_v, trans_v, gsem):
        wid = lax.axis_index("s") * _NC + lax.axis_index("c")
        qvecs = [lax.iota(jnp.int32, _L) + qg * _L for qg in range(_TB // _L)]

        def body(g, carry):
            pltpu.sync_copy(idx4_hbm.at[g, wid], idx_t)
            copies = [
                pltpu.async_copy(
                    tab_hbm.at[idx_t.at[p]], rows_v.at[p], gsem)
                for p in range(_TS)
            ]
            for c in copies:
                c.wait()

            def tbody(ps, tcarry):
                psvec = jnp.full((_L,), ps, jnp.int32)
                for di in range(ndt):
                    divec = jnp.full((_L,), di, jnp.int32)
                    for dp in range(_TS):
                        dvec = jnp.full((_L,), di * _TS + dp, jnp.int32)
                        dpvec = jnp.full((_L,), dp, jnp.int32)
                        for qv in qvecs:
                            vals = plsc.load_gather(rows_v, [psvec, qv, dvec])
                            plsc.store_scatter(
                                trans_v, [psvec, divec, dpvec, qv], vals)
                return tcarry

            lax.fori_loop(0, _TS, tbody, 0)
            pltpu.sync_copy(trans_v, out5_hbm.at[pl.ds(g * _TS, _TS), :, wid])
            return carry

        lax.fori_loop(0, nst, body, 0)

    return k


def kernel(x, w):
    bsz, seq = x.shape
    assert bsz == _NW * _TB and seq % _TS == 0 and _D % _TS == 0
    nst = seq // _TS
    ndt = _D // _TS
    x4 = (x.astype(jnp.int32).T
          .reshape(nst, _TS, _NW, _TB).transpose(0, 2, 1, 3))
    out5 = _gather_call(bsz, seq)(x4, w)
    return out5.transpose(2, 4, 0, 1, 3).reshape(bsz, seq, _D)
